# baseline (speedup 1.0000x reference)
import functools

import jax
import jax.numpy as jnp
from jax.experimental import pallas as pl

_N = 4096
_BMA = 512
_BMB = 1024


def _deg_xs_body(d_ref, x_ref, deg_ref, xs_ref):
    d = jnp.max(d_ref[...], axis=1, keepdims=True)
    deg_ref[...] = d
    xs_ref[...] = (x_ref[...] * jax.lax.rsqrt(d)).astype(jnp.bfloat16)


def _conv_body(a_ref, xs_ref, deg_ref, w_ref, b_ref, o_ref):
    a = a_ref[...]
    ab = a > 0
    af = ab.astype(jnp.bfloat16)
    agg = jnp.dot(af, xs_ref[...], preferred_element_type=jnp.float32)

    iota = jax.lax.broadcasted_iota(jnp.int32, a.shape, 1)
    masked = jnp.where(ab, iota, _N)
    first = jnp.min(masked, axis=1, keepdims=True)
    first = jnp.where(first >= _N, 0, first)
    onehot = (iota == first).astype(jnp.float32)
    dfirst = jnp.dot(onehot, deg_ref[...], preferred_element_type=jnp.float32)
    c = jax.lax.rsqrt(dfirst)

    z = jax.lax.dot_general(
        agg, w_ref[...], (((1,), (1,)), ((), ())),
        preferred_element_type=jnp.float32)
    z = z * c + b_ref[...]
    o_ref[...] = jnp.where(z >= 0.0, z, 0.01 * z)


@jax.jit
def kernel(D, X, A, W, b):
    n, in_ch = X.shape
    out_ch = W.shape[0]

    deg, xs = pl.pallas_call(
        _deg_xs_body,
        grid=(n // _BMA,),
        in_specs=[
            pl.BlockSpec((_BMA, n), lambda i: (i, 0)),
            pl.BlockSpec((_BMA, in_ch), lambda i: (i, 0)),
        ],
        out_specs=[
            pl.BlockSpec((_BMA, 1), lambda i: (i, 0)),
            pl.BlockSpec((_BMA, in_ch), lambda i: (i, 0)),
        ],
        out_shape=[
            jax.ShapeDtypeStruct((n, 1), jnp.float32),
            jax.ShapeDtypeStruct((n, in_ch), jnp.bfloat16),
        ],
    )(D, X)

    out = pl.pallas_call(
        _conv_body,
        grid=(n // _BMB,),
        in_specs=[
            pl.BlockSpec((_BMB, n), lambda i: (i, 0)),
            pl.BlockSpec((n, in_ch), lambda i: (0, 0)),
            pl.BlockSpec((n, 1), lambda i: (0, 0)),
            pl.BlockSpec((out_ch, in_ch), lambda i: (0, 0)),
            pl.BlockSpec((1, out_ch), lambda i: (0, 0)),
        ],
        out_specs=pl.BlockSpec((_BMB, out_ch), lambda i: (i, 0)),
        out_shape=jax.ShapeDtypeStruct((n, out_ch), jnp.float32),
    )(A, xs, deg, W, b.reshape(1, out_ch))
    return out


# speedup vs baseline: 1.0988x; 1.0988x over previous
import jax
import jax.numpy as jnp
from jax.experimental import pallas as pl
from jax.experimental.pallas import tpu as pltpu

_N = 4096
_BK = 256


def _fused_body(d_ref, x_ref, a_ref, w_ref, b_ref, o_ref,
                acc_ref, gfirst_ref, gval_ref):
    k = pl.program_id(0)
    nsteps = pl.num_programs(0)

    @pl.when(k == 0)
    def _init():
        acc_ref[...] = jnp.zeros_like(acc_ref)
        gfirst_ref[...] = jnp.full_like(gfirst_ref, _N)
        gval_ref[...] = jnp.ones_like(gval_ref)

    d = jnp.max(d_ref[...], axis=1, keepdims=True)
    xs = (x_ref[...] * jax.lax.rsqrt(d)).astype(jnp.bfloat16)

    a = a_ref[...]
    ab = a > 0
    af = ab.astype(jnp.bfloat16)
    acc_ref[...] += jnp.dot(af, xs, preferred_element_type=jnp.float32)

    iota = jax.lax.broadcasted_iota(jnp.int32, a.shape, 1) + k * _BK
    masked = jnp.where(ab, iota, _N)
    lmin = jnp.min(masked, axis=1, keepdims=True)
    onehot = (iota == lmin).astype(jnp.float32)
    lval = jnp.dot(onehot, d, preferred_element_type=jnp.float32)
    upd = lmin < gfirst_ref[...]
    gval_ref[...] = jnp.where(upd, lval, gval_ref[...])
    gfirst_ref[...] = jnp.where(upd, lmin, gfirst_ref[...])

    @pl.when(k == nsteps - 1)
    def _epilogue():
        c = jax.lax.rsqrt(gval_ref[...])
        z = jax.lax.dot_general(
            acc_ref[...], w_ref[...], (((1,), (1,)), ((), ())),
            preferred_element_type=jnp.float32)
        z = z * c + b_ref[...]
        o_ref[...] = jnp.where(z >= 0.0, z, 0.01 * z)


@jax.jit
def kernel(D, X, A, W, b):
    n, in_ch = X.shape
    out_ch = W.shape[0]

    out = pl.pallas_call(
        _fused_body,
        grid=(n // _BK,),
        in_specs=[
            pl.BlockSpec((_BK, n), lambda k: (k, 0)),
            pl.BlockSpec((_BK, in_ch), lambda k: (k, 0)),
            pl.BlockSpec((n, _BK), lambda k: (0, k)),
            pl.BlockSpec((out_ch, in_ch), lambda k: (0, 0)),
            pl.BlockSpec((1, out_ch), lambda k: (0, 0)),
        ],
        out_specs=pl.BlockSpec((n, out_ch), lambda k: (0, 0)),
        out_shape=jax.ShapeDtypeStruct((n, out_ch), jnp.float32),
        scratch_shapes=[
            pltpu.VMEM((n, out_ch), jnp.float32),
            pltpu.VMEM((n, 1), jnp.int32),
            pltpu.VMEM((n, 1), jnp.float32),
        ],
    )(D, X, A, W, b.reshape(1, out_ch))
    return out


# speedup vs baseline: 1.2181x; 1.1086x over previous
import jax
import jax.numpy as jnp
from jax.experimental import pallas as pl
from jax.experimental.pallas import tpu as pltpu

_N = 4096
_BK = 512


def _fused_body(d_ref, x_ref, a_ref, w_ref, b_ref, o_ref,
                acc_ref, gfirst_ref, gval_ref):
    k = pl.program_id(0)
    nsteps = pl.num_programs(0)

    @pl.when(k == 0)
    def _init():
        acc_ref[...] = jnp.zeros_like(acc_ref)
        gfirst_ref[...] = jnp.full_like(gfirst_ref, _N)
        gval_ref[...] = jnp.ones_like(gval_ref)

    d = jnp.max(d_ref[...], axis=1, keepdims=True)
    xs = (x_ref[...] * jax.lax.rsqrt(d)).astype(jnp.bfloat16)

    a = a_ref[...]
    ab = a > 0
    af = ab.astype(jnp.bfloat16)
    acc_ref[...] += jnp.dot(af, xs, preferred_element_type=jnp.float32)

    iota = jax.lax.broadcasted_iota(jnp.int32, a.shape, 1) + k * _BK
    masked = jnp.where(ab, iota, _N)
    lmin = jnp.min(masked, axis=1, keepdims=True)
    onehot = (iota == lmin).astype(jnp.float32)
    lval = jnp.dot(onehot, d, preferred_element_type=jnp.float32)
    upd = lmin < gfirst_ref[...]
    gval_ref[...] = jnp.where(upd, lval, gval_ref[...])
    gfirst_ref[...] = jnp.where(upd, lmin, gfirst_ref[...])

    @pl.when(k == nsteps - 1)
    def _epilogue():
        c = jax.lax.rsqrt(gval_ref[...])
        z = jax.lax.dot_general(
            acc_ref[...], w_ref[...], (((1,), (1,)), ((), ())),
            preferred_element_type=jnp.float32)
        z = z * c + b_ref[...]
        o_ref[...] = jnp.where(z >= 0.0, z, 0.01 * z)


@jax.jit
def kernel(D, X, A, W, b):
    n, in_ch = X.shape
    out_ch = W.shape[0]

    out = pl.pallas_call(
        _fused_body,
        grid=(n // _BK,),
        in_specs=[
            pl.BlockSpec((_BK, n), lambda k: (k, 0)),
            pl.BlockSpec((_BK, in_ch), lambda k: (k, 0)),
            pl.BlockSpec((n, _BK), lambda k: (0, k)),
            pl.BlockSpec((out_ch, in_ch), lambda k: (0, 0)),
            pl.BlockSpec((1, out_ch), lambda k: (0, 0)),
        ],
        out_specs=pl.BlockSpec((n, out_ch), lambda k: (0, 0)),
        out_shape=jax.ShapeDtypeStruct((n, out_ch), jnp.float32),
        scratch_shapes=[
            pltpu.VMEM((n, out_ch), jnp.float32),
            pltpu.VMEM((n, 1), jnp.int32),
            pltpu.VMEM((n, 1), jnp.float32),
        ],
    )(D, X, A, W, b.reshape(1, out_ch))
    return out
